# Initial kernel scaffold; baseline (speedup 1.0000x reference)
import jax
import jax.numpy as jnp
from jax.experimental import pallas as pl


def kernel(pred, gt):
    raise NotImplementedError("write your pallas kernel here")


# speedup vs baseline: 2.5948x; 2.5948x over previous
import math

import jax
import jax.numpy as jnp
from jax.experimental import pallas as pl
from jax.experimental.pallas import tpu as pltpu

_N = 20000
_M = 2000
_TN = 1024
_NPAD = 20480
_NT = _NPAD // _TN
_MPAD = 2048
_R2 = (2.5 * math.log(3.0)) ** 2 - 1e-12


def _loss_kernel(px_ref, py_ref, pc_ref, conf_ref, tx_ref, ty_ref, tc_ref,
                 obj_ref, reg_ref, bd2_ref, bidx_ref, rmin_ref):
    i = pl.program_id(1)

    pxr = px_ref[0, 0]
    pyr = py_ref[0, 0]
    pcr = pc_ref[0, 0]
    txc = tx_ref[0]
    tyc = ty_ref[0]
    tcc = tc_ref[0]

    dx = txc - pxr
    dy = tyc - pyr
    d2 = dx * dx + dy * dy

    rmin_ref[i] = jnp.min(d2, axis=0, keepdims=True)

    d2m = jnp.where(tcc == pcr, d2, jnp.inf)
    tmin = jnp.min(d2m, axis=1, keepdims=True)
    lane = jax.lax.broadcasted_iota(jnp.int32, (_MPAD, _TN), 1)
    lidx = jnp.min(jnp.where(d2m == tmin, lane, _TN), axis=1, keepdims=True)
    gidx = i * _TN + lidx

    @pl.when(i == 0)
    def _():
        bd2_ref[...] = tmin
        bidx_ref[...] = gidx

    @pl.when(i > 0)
    def _():
        upd = tmin < bd2_ref[...]
        bd2_ref[...] = jnp.where(upd, tmin, bd2_ref[...])
        bidx_ref[...] = jnp.where(upd, gidx, bidx_ref[...])

    @pl.when(i == _NT - 1)
    def _():
        has = bd2_ref[...] <= _R2
        bidx = bidx_ref[...]

        def body(j, carry):
            osum, rsum = carry
            idxs = j * _TN + jax.lax.broadcasted_iota(jnp.int32, (1, _TN), 1)
            match = (bidx == idxs) & has
            hit = jnp.max(jnp.where(match, 1.0, 0.0), axis=0, keepdims=True)
            rm = rmin_ref[j]
            s = jnp.sqrt(rm + 1e-12) * (1.0 / 2.5)
            t = jnp.exp(-s)
            rowsc = 2.0 * t / (1.0 + t)
            cf = conf_ref[0, j]
            sp = jnp.maximum(cf, 0.0) + jnp.log1p(jnp.exp(-jnp.abs(cf)))
            real = idxs < _N
            osum = osum + jnp.sum(jnp.where(real, sp - hit * cf, 0.0))
            rsum = rsum + jnp.sum(hit * rowsc)
            return (osum, rsum)

        osum, rsum = jax.lax.fori_loop(0, _NT, body,
                                       (jnp.float32(0.0), jnp.float32(0.0)))
        obj_ref[...] = jnp.full((1, 8, 128), osum / _N, dtype=jnp.float32)
        reg_ref[...] = jnp.full((1, 8, 128), 1.0 - rsum / _N, dtype=jnp.float32)


def _run(px, py, pc, cf, tx, ty, tc):
    B = px.shape[0]
    grid = (B, _NT)
    obj, reg = pl.pallas_call(
        _loss_kernel,
        grid=grid,
        in_specs=[
            pl.BlockSpec((1, 1, 1, _TN), lambda b, i: (b, i, 0, 0)),
            pl.BlockSpec((1, 1, 1, _TN), lambda b, i: (b, i, 0, 0)),
            pl.BlockSpec((1, 1, 1, _TN), lambda b, i: (b, i, 0, 0)),
            pl.BlockSpec((1, _NT, 1, _TN), lambda b, i: (b, 0, 0, 0)),
            pl.BlockSpec((1, _MPAD, 1), lambda b, i: (b, 0, 0)),
            pl.BlockSpec((1, _MPAD, 1), lambda b, i: (b, 0, 0)),
            pl.BlockSpec((1, _MPAD, 1), lambda b, i: (b, 0, 0)),
        ],
        out_specs=[
            pl.BlockSpec((1, 8, 128), lambda b, i: (b, 0, 0)),
            pl.BlockSpec((1, 8, 128), lambda b, i: (b, 0, 0)),
        ],
        out_shape=[
            jax.ShapeDtypeStruct((B, 8, 128), jnp.float32),
            jax.ShapeDtypeStruct((B, 8, 128), jnp.float32),
        ],
        scratch_shapes=[
            pltpu.VMEM((_MPAD, 1), jnp.float32),
            pltpu.VMEM((_MPAD, 1), jnp.int32),
            pltpu.VMEM((_NT, 1, _TN), jnp.float32),
        ],
        compiler_params=pltpu.CompilerParams(
            dimension_semantics=("arbitrary", "arbitrary")),
    )(px, py, pc, cf, tx, ty, tc)
    return obj[:, 0, 0], reg[:, 0, 0]


def kernel(pred, gt):
    B = pred.shape[0]
    padn = _NPAD - _N

    def padp(x, val):
        x = jnp.pad(x, ((0, 0), (0, padn)), constant_values=val)
        return x.reshape(B, _NT, 1, _TN)

    pc = padp(pred[..., 0], -1.0)
    px = padp(pred[..., 1], 1e9)
    py = padp(pred[..., 2], 1e9)
    cf = padp(pred[..., 3], 0.0)

    padm = _MPAD - _M

    def padt(x, val):
        return jnp.pad(x, ((0, 0), (0, padm)), constant_values=val)[..., None]

    tc = padt(gt[..., 0], -2.0)
    tx = padt(gt[..., 1], 2e9)
    ty = padt(gt[..., 2], 2e9)

    obj, reg = _run(px, py, pc, cf, tx, ty, tc)
    return (jnp.mean(obj), jnp.mean(reg))


# speedup vs baseline: 3.4660x; 1.3358x over previous
import functools
import math

import jax
import jax.numpy as jnp
from jax import lax
from jax.experimental import pallas as pl
from jax.experimental.pallas import tpu as pltpu
from jax.experimental.pallas import tpu_sc as plsc

_N = 20000
_M = 2000
_B = 4
_TN = 1024
_NPAD = 20480
_NT = _NPAD // _TN
_MPAD = 2048
_R2 = (2.5 * math.log(3.0)) ** 2 - 1e-12


def _tc_kernel(px_ref, py_ref, pc_ref, conf_ref, tx_ref, ty_ref, tc_ref,
               midx_ref, rowsc_ref, osum_ref, bd2_ref, bidx_ref):
    i = pl.program_id(1)

    pxr = px_ref[0, 0]
    pyr = py_ref[0, 0]
    pcr = pc_ref[0, 0]
    txc = tx_ref[0]
    tyc = ty_ref[0]
    tcc = tc_ref[0]

    dx = txc - pxr
    dy = tyc - pyr
    d2 = dx * dx + dy * dy

    rm = jnp.min(d2, axis=0, keepdims=True)
    s = jnp.sqrt(rm + 1e-12) * (1.0 / 2.5)
    t = jnp.exp(-s)
    rowsc_ref[0, 0] = 2.0 * t / (1.0 + t)

    cf = conf_ref[0, 0]
    lane1 = jax.lax.broadcasted_iota(jnp.int32, (1, _TN), 1)
    real = (i * _TN + lane1) < _N
    sp = jnp.maximum(cf, 0.0) + jnp.log1p(jnp.exp(-jnp.abs(cf)))
    part = jnp.sum(jnp.where(real, sp, 0.0))

    d2m = jnp.where(tcc == pcr, d2, jnp.inf)
    tmin = jnp.min(d2m, axis=1, keepdims=True)
    lane = jax.lax.broadcasted_iota(jnp.int32, (_MPAD, _TN), 1)
    lidx = jnp.min(jnp.where(d2m == tmin, lane, _TN), axis=1, keepdims=True)
    gidx = i * _TN + lidx

    @pl.when(i == 0)
    def _():
        bd2_ref[...] = tmin
        bidx_ref[...] = gidx
        osum_ref[...] = jnp.full((1, 8, 128), part, dtype=jnp.float32)

    @pl.when(i > 0)
    def _():
        upd = tmin < bd2_ref[...]
        bd2_ref[...] = jnp.where(upd, tmin, bd2_ref[...])
        bidx_ref[...] = jnp.where(upd, gidx, bidx_ref[...])
        osum_ref[...] = osum_ref[...] + part

    @pl.when(i == _NT - 1)
    def _():
        has = bd2_ref[...] <= _R2
        midx_ref[0] = jnp.where(has, bidx_ref[...], _NPAD - 1)


def _tc_stage(px, py, pc, cf, tx, ty, tc):
    grid = (_B, _NT)
    return pl.pallas_call(
        _tc_kernel,
        grid=grid,
        in_specs=[
            pl.BlockSpec((1, 1, 1, _TN), lambda b, i: (b, i, 0, 0)),
            pl.BlockSpec((1, 1, 1, _TN), lambda b, i: (b, i, 0, 0)),
            pl.BlockSpec((1, 1, 1, _TN), lambda b, i: (b, i, 0, 0)),
            pl.BlockSpec((1, 1, 1, _TN), lambda b, i: (b, i, 0, 0)),
            pl.BlockSpec((1, _MPAD, 1), lambda b, i: (b, 0, 0)),
            pl.BlockSpec((1, _MPAD, 1), lambda b, i: (b, 0, 0)),
            pl.BlockSpec((1, _MPAD, 1), lambda b, i: (b, 0, 0)),
        ],
        out_specs=[
            pl.BlockSpec((1, _MPAD, 1), lambda b, i: (b, 0, 0)),
            pl.BlockSpec((1, 1, 1, _TN), lambda b, i: (b, i, 0, 0)),
            pl.BlockSpec((1, 8, 128), lambda b, i: (b, 0, 0)),
        ],
        out_shape=[
            jax.ShapeDtypeStruct((_B, _MPAD, 1), jnp.int32),
            jax.ShapeDtypeStruct((_B, _NT, 1, _TN), jnp.float32),
            jax.ShapeDtypeStruct((_B, 8, 128), jnp.float32),
        ],
        scratch_shapes=[
            pltpu.VMEM((_MPAD, 1), jnp.float32),
            pltpu.VMEM((_MPAD, 1), jnp.int32),
        ],
        compiler_params=pltpu.CompilerParams(
            dimension_semantics=("arbitrary", "arbitrary")),
    )(px, py, pc, cf, tx, ty, tc)


_NW = 32
_TPW = (2 * _MPAD * 2) // _NW


def _sc_kernel(idxs_hbm, idxg_hbm, mval_hbm, conf_hbm, rowsc_hbm, out_hbm,
               idxs_v, idxg_v, m_v, w_v, cf_v, rs_v, out_v, win_sh, sem):
    c = lax.axis_index("c")
    s = lax.axis_index("s")
    r = c * 16 + s

    pltpu.sync_copy(idxs_hbm.at[r], idxs_v)
    pltpu.sync_copy(idxg_hbm.at[r], idxg_v)
    pltpu.sync_copy(mval_hbm.at[r], m_v)

    for j in range(2):
        pltpu.async_copy(m_v.at[j], win_sh.at[idxs_v.at[j]], sem).wait()
    plsc.subcore_barrier()
    for j in range(2):
        pltpu.async_copy(win_sh.at[idxs_v.at[j]], w_v.at[j], sem).wait()
        pltpu.async_copy(conf_hbm.at[idxg_v.at[j]], cf_v.at[j], sem).wait()
        pltpu.async_copy(rowsc_hbm.at[idxg_v.at[j]], rs_v.at[j], sem).wait()

    acc_c = jnp.zeros((16,), jnp.float32)
    acc_r = jnp.zeros((16,), jnp.float32)
    for j in range(2):
        for k in range(8):
            sl = pl.ds(k * 16, 16)
            keep = w_v[j, sl] == m_v[j, sl]
            acc_c = acc_c + jnp.where(keep, cf_v[j, sl], 0.0)
            acc_r = acc_r + jnp.where(keep, rs_v[j, sl], 0.0)

    out_v[0, :] = acc_c
    out_v[1, :] = acc_r
    pltpu.sync_copy(out_v, out_hbm.at[r])


@functools.partial(
    pl.kernel,
    mesh=plsc.VectorSubcoreMesh(core_axis_name="c", subcore_axis_name="s"),
    out_type=jax.ShapeDtypeStruct((_NW, 2, 16), jnp.float32),
    scratch_types=[
        pltpu.VMEM((2, 128), jnp.int32),
        pltpu.VMEM((2, 128), jnp.int32),
        pltpu.VMEM((2, 128), jnp.int32),
        pltpu.VMEM((2, 128), jnp.int32),
        pltpu.VMEM((2, 128), jnp.float32),
        pltpu.VMEM((2, 128), jnp.float32),
        pltpu.VMEM((2, 16), jnp.float32),
        pltpu.VMEM_SHARED((2 * _NPAD,), jnp.int32),
        pltpu.SemaphoreType.DMA,
    ],
)
def _sc_stage(idxs_hbm, idxg_hbm, mval_hbm, conf_hbm, rowsc_hbm, out_hbm,
              idxs_v, idxg_v, m_v, w_v, cf_v, rs_v, out_v, win_sh, sem):
    _sc_kernel(idxs_hbm, idxg_hbm, mval_hbm, conf_hbm, rowsc_hbm, out_hbm,
               idxs_v, idxg_v, m_v, w_v, cf_v, rs_v, out_v, win_sh, sem)


def kernel(pred, gt):
    B = pred.shape[0]
    padn = _NPAD - _N

    def padp(x, val):
        x = jnp.pad(x, ((0, 0), (0, padn)), constant_values=val)
        return x.reshape(B, _NT, 1, _TN)

    pc = padp(pred[..., 0], -1.0)
    px = padp(pred[..., 1], 1e9)
    py = padp(pred[..., 2], 1e9)
    cf = padp(pred[..., 3], 0.0)

    padm = _MPAD - _M

    def padt(x, val):
        return jnp.pad(x, ((0, 0), (0, padm)), constant_values=val)[..., None]

    tc = padt(gt[..., 0], -2.0)
    tx = padt(gt[..., 1], 2e9)
    ty = padt(gt[..., 2], 2e9)

    midx, rowsc, osum = _tc_stage(px, py, pc, cf, tx, ty, tc)

    m2 = midx[:, :, 0]
    barr = jnp.arange(B, dtype=jnp.int32)[:, None]
    idx_sc = ((barr % 2) * _NPAD + m2).reshape(_NW, 2, 128)
    idx_g = (barr * _NPAD + m2).reshape(_NW, 2, 128)
    mval = jnp.tile(jnp.arange(2 * _MPAD, dtype=jnp.int32), (2,))
    mval = mval.reshape(_NW, 2, 128)
    conf_flat = cf.reshape(B * _NPAD)
    rowsc_flat = rowsc.reshape(B * _NPAD)

    out = _sc_stage(idx_sc, idx_g, mval, conf_flat, rowsc_flat)

    o = out.reshape(2, 2, 8, 2, 16)
    mconf = o[:, :, :, 0, :].sum(axis=(2, 3)).reshape(B)
    mrow = o[:, :, :, 1, :].sum(axis=(2, 3)).reshape(B)
    obj_b = (osum[:, 0, 0] - mconf) / _N
    reg_b = 1.0 - mrow / _N
    return (jnp.mean(obj_b), jnp.mean(reg_b))


# speedup vs baseline: 3.6205x; 1.0446x over previous
import functools
import math

import jax
import jax.numpy as jnp
from jax import lax
from jax.experimental import pallas as pl
from jax.experimental.pallas import tpu as pltpu
from jax.experimental.pallas import tpu_sc as plsc

_N = 20000
_M = 2000
_B = 4
_TN = 1024
_NPAD = 20480
_NT = _NPAD // _TN
_MPAD = 2048
_R2 = (2.5 * math.log(3.0)) ** 2 - 1e-12


def _tc_kernel(px_ref, py_ref, pc_ref, conf_ref, tx_ref, ty_ref, tc_ref,
               midx_ref, rowsc_ref, osum_ref, bd2_ref, bidx_ref):
    i = pl.program_id(1)

    pxr = px_ref[0, 0]
    pyr = py_ref[0, 0]
    pcr = pc_ref[0, 0]
    txc = tx_ref[0]
    tyc = ty_ref[0]
    tcc = tc_ref[0]

    dx = txc - pxr
    dy = tyc - pyr
    d2 = dx * dx + dy * dy

    rm = jnp.min(d2, axis=0, keepdims=True)
    s = jnp.sqrt(rm + 1e-12) * (1.0 / 2.5)
    t = jnp.exp(-s)
    rowsc_ref[0, 0] = 2.0 * t / (1.0 + t)

    cf = conf_ref[0, 0]
    lane1 = jax.lax.broadcasted_iota(jnp.int32, (1, _TN), 1)
    real = (i * _TN + lane1) < _N
    sp = jnp.maximum(cf, 0.0) + jnp.log1p(jnp.exp(-jnp.abs(cf)))
    part = jnp.sum(jnp.where(real, sp, 0.0))

    d2m = jnp.where(tcc == pcr, d2, jnp.inf)
    lane = jax.lax.broadcasted_iota(jnp.int32, (_MPAD, _TN), 1)
    ip = (jax.lax.bitcast_convert_type(d2m, jnp.int32)
          & jnp.int32(-1024)) | lane
    m = jnp.min(ip, axis=1, keepdims=True)
    lidx = m & 1023
    tmin = jax.lax.bitcast_convert_type(m & jnp.int32(-1024), jnp.float32)
    gidx = i * _TN + lidx

    @pl.when(i == 0)
    def _():
        bd2_ref[...] = tmin
        bidx_ref[...] = gidx
        osum_ref[...] = jnp.full((1, 8, 128), part, dtype=jnp.float32)

    @pl.when(i > 0)
    def _():
        upd = tmin < bd2_ref[...]
        bd2_ref[...] = jnp.where(upd, tmin, bd2_ref[...])
        bidx_ref[...] = jnp.where(upd, gidx, bidx_ref[...])
        osum_ref[...] = osum_ref[...] + part

    @pl.when(i == _NT - 1)
    def _():
        has = bd2_ref[...] <= _R2
        midx_ref[0] = jnp.where(has, bidx_ref[...], _NPAD - 1)


def _tc_stage(px, py, pc, cf, tx, ty, tc):
    grid = (_B, _NT)
    return pl.pallas_call(
        _tc_kernel,
        grid=grid,
        in_specs=[
            pl.BlockSpec((1, 1, 1, _TN), lambda b, i: (b, i, 0, 0)),
            pl.BlockSpec((1, 1, 1, _TN), lambda b, i: (b, i, 0, 0)),
            pl.BlockSpec((1, 1, 1, _TN), lambda b, i: (b, i, 0, 0)),
            pl.BlockSpec((1, 1, 1, _TN), lambda b, i: (b, i, 0, 0)),
            pl.BlockSpec((1, _MPAD, 1), lambda b, i: (b, 0, 0)),
            pl.BlockSpec((1, _MPAD, 1), lambda b, i: (b, 0, 0)),
            pl.BlockSpec((1, _MPAD, 1), lambda b, i: (b, 0, 0)),
        ],
        out_specs=[
            pl.BlockSpec((1, _MPAD, 1), lambda b, i: (b, 0, 0)),
            pl.BlockSpec((1, 1, 1, _TN), lambda b, i: (b, i, 0, 0)),
            pl.BlockSpec((1, 8, 128), lambda b, i: (b, 0, 0)),
        ],
        out_shape=[
            jax.ShapeDtypeStruct((_B, _MPAD, 1), jnp.int32),
            jax.ShapeDtypeStruct((_B, _NT, 1, _TN), jnp.float32),
            jax.ShapeDtypeStruct((_B, 8, 128), jnp.float32),
        ],
        scratch_shapes=[
            pltpu.VMEM((_MPAD, 1), jnp.float32),
            pltpu.VMEM((_MPAD, 1), jnp.int32),
        ],
        compiler_params=pltpu.CompilerParams(
            dimension_semantics=("arbitrary", "arbitrary")),
    )(px, py, pc, cf, tx, ty, tc)


_NW = 32
_TPW = (2 * _MPAD * 2) // _NW


def _sc_kernel(idxs_hbm, idxg_hbm, mval_hbm, conf_hbm, rowsc_hbm, out_hbm,
               idxs_v, idxg_v, m_v, w_v, cf_v, rs_v, out_v, win_sh, sem):
    c = lax.axis_index("c")
    s = lax.axis_index("s")
    r = c * 16 + s

    pltpu.sync_copy(idxs_hbm.at[r], idxs_v)
    pltpu.sync_copy(idxg_hbm.at[r], idxg_v)
    pltpu.sync_copy(mval_hbm.at[r], m_v)

    for j in range(2):
        pltpu.async_copy(m_v.at[j], win_sh.at[idxs_v.at[j]], sem).wait()
    plsc.subcore_barrier()
    for j in range(2):
        pltpu.async_copy(win_sh.at[idxs_v.at[j]], w_v.at[j], sem).wait()
        pltpu.async_copy(conf_hbm.at[idxg_v.at[j]], cf_v.at[j], sem).wait()
        pltpu.async_copy(rowsc_hbm.at[idxg_v.at[j]], rs_v.at[j], sem).wait()

    acc_c = jnp.zeros((16,), jnp.float32)
    acc_r = jnp.zeros((16,), jnp.float32)
    for j in range(2):
        for k in range(8):
            sl = pl.ds(k * 16, 16)
            keep = w_v[j, sl] == m_v[j, sl]
            acc_c = acc_c + jnp.where(keep, cf_v[j, sl], 0.0)
            acc_r = acc_r + jnp.where(keep, rs_v[j, sl], 0.0)

    out_v[0, :] = acc_c
    out_v[1, :] = acc_r
    pltpu.sync_copy(out_v, out_hbm.at[r])


@functools.lru_cache(maxsize=1)
def _sc_stage_fn():
    return functools.partial(
        pl.kernel,
        mesh=plsc.VectorSubcoreMesh(core_axis_name="c", subcore_axis_name="s"),
        out_type=jax.ShapeDtypeStruct((_NW, 2, 16), jnp.float32),
        scratch_types=[
            pltpu.VMEM((2, 128), jnp.int32),
            pltpu.VMEM((2, 128), jnp.int32),
            pltpu.VMEM((2, 128), jnp.int32),
            pltpu.VMEM((2, 128), jnp.int32),
            pltpu.VMEM((2, 128), jnp.float32),
            pltpu.VMEM((2, 128), jnp.float32),
            pltpu.VMEM((2, 16), jnp.float32),
            pltpu.VMEM_SHARED((2 * _NPAD,), jnp.int32),
            pltpu.SemaphoreType.DMA,
        ],
    )(_sc_kernel)


def _sc_stage(idx_sc, idx_g, mval, conf_flat, rowsc_flat):
    return _sc_stage_fn()(idx_sc, idx_g, mval, conf_flat, rowsc_flat)


def kernel(pred, gt):
    B = pred.shape[0]
    padn = _NPAD - _N

    def padp(x, val):
        x = jnp.pad(x, ((0, 0), (0, padn)), constant_values=val)
        return x.reshape(B, _NT, 1, _TN)

    pc = padp(pred[..., 0], -1.0)
    px = padp(pred[..., 1], 1e9)
    py = padp(pred[..., 2], 1e9)
    cf = padp(pred[..., 3], 0.0)

    padm = _MPAD - _M

    def padt(x, val):
        return jnp.pad(x, ((0, 0), (0, padm)), constant_values=val)[..., None]

    tc = padt(gt[..., 0], -2.0)
    tx = padt(gt[..., 1], 2e9)
    ty = padt(gt[..., 2], 2e9)

    midx, rowsc, osum = _tc_stage(px, py, pc, cf, tx, ty, tc)

    m2 = jnp.minimum(midx[:, :, 0], _NPAD - 1)
    barr = jnp.arange(B, dtype=jnp.int32)[:, None]
    idx_sc = ((barr % 2) * _NPAD + m2).reshape(_NW, 2, 128)
    idx_g = (barr * _NPAD + m2).reshape(_NW, 2, 128)
    mval = jnp.tile(jnp.arange(2 * _MPAD, dtype=jnp.int32), (2,))
    mval = mval.reshape(_NW, 2, 128)
    conf_flat = cf.reshape(B * _NPAD)
    rowsc_flat = rowsc.reshape(B * _NPAD)

    out = _sc_stage(idx_sc, idx_g, mval, conf_flat, rowsc_flat)

    o = out.reshape(2, 2, 8, 2, 16)
    mconf = o[:, :, :, 0, :].sum(axis=(2, 3)).reshape(B)
    mrow = o[:, :, :, 1, :].sum(axis=(2, 3)).reshape(B)
    obj_b = (osum[:, 0, 0] - mconf) / _N
    reg_b = 1.0 - mrow / _N
    return (jnp.mean(obj_b), jnp.mean(reg_b))
